# baseline (speedup 1.0000x reference)
import functools

import jax
import jax.numpy as jnp
from jax import lax
from jax.experimental import pallas as pl
from jax.experimental.pallas import tpu as pltpu
from jax.experimental.pallas import tpu_sc as plsc

_N = 10000
_E = 320000
_D = 128

_NC = 2
_NS = 16
_NW = _NC * _NS
_CHUNK = 128
_NCHUNK = 80
_NQ = 4
_QCH = _NCHUNK // _NQ
_QPAIRS = _QCH // 2
_EPW = _NCHUNK * _CHUNK
_EPAD = _NW * _EPW
_NPAD = 10240
_RPS = _NPAD // _NS


def _sc_aggregate(x, src3, dst3):
    mesh = plsc.VectorSubcoreMesh(core_axis_name="c", subcore_axis_name="s")

    @functools.partial(
        pl.kernel,
        mesh=mesh,
        out_type=jax.ShapeDtypeStruct((2, _NPAD, _D), jnp.float32),
        scratch_types=[
            pltpu.VMEM((_QCH, _CHUNK), jnp.int32),
            pltpu.VMEM((_QCH, _CHUNK), jnp.int32),
            pltpu.VMEM((_QCH, _CHUNK), jnp.int32),
            pltpu.VMEM((_QCH, _CHUNK), jnp.int32),
            pltpu.VMEM((_CHUNK, _D), jnp.float32),
            pltpu.VMEM((_CHUNK, _D), jnp.float32),
            pltpu.VMEM_SHARED((_NPAD, _D), jnp.float32),
            pltpu.SemaphoreType.DMA,
            pltpu.SemaphoreType.DMA,
            pltpu.SemaphoreType.DMA,
            pltpu.SemaphoreType.DMA,
        ],
    )
    def agg(x_hbm, src_hbm, dst_hbm, out_hbm, srcs0, srcs1, dsts0, dsts1,
            rows0, rows1, acc_s, semi0, semi1, semg0, semg1):
        c = lax.axis_index("c")
        s = lax.axis_index("s")
        wid = c * _NS + s
        sbufs = (srcs0, srcs1)
        dbufs = (dsts0, dsts1)
        isems = (semi0, semi1)

        def start_idx(q):
            b = q % 2
            pltpu.async_copy(src_hbm.at[wid * _NQ + q], sbufs[b], isems[b])
            pltpu.async_copy(dst_hbm.at[wid * _NQ + q], dbufs[b], isems[b])

        def wait_idx(q):
            b = q % 2
            pltpu.make_async_copy(src_hbm.at[wid * _NQ + q], sbufs[b], isems[b]).wait()
            pltpu.make_async_copy(dst_hbm.at[wid * _NQ + q], dbufs[b], isems[b]).wait()

        start_idx(0)

        def zstore(i, carry):
            rows0[i // (_D // 16), pl.ds((i % (_D // 16)) * 16, 16)] = jnp.zeros(
                (16,), jnp.float32)
            return carry

        lax.fori_loop(0, _CHUNK * (_D // 16), zstore, None)

        def zcopy(j, carry):
            pltpu.sync_copy(rows0, acc_s.at[pl.ds(s * _RPS + j * _CHUNK, _CHUNK)])
            return carry

        lax.fori_loop(0, _RPS // _CHUNK, zcopy, None)

        wait_idx(0)
        plsc.subcore_barrier()

        def wait_g(buf, sem):
            pltpu.make_async_copy(x_hbm.at[pl.ds(0, _CHUNK)], buf, sem).wait()

        for q in range(_NQ):
            src_q = sbufs[q % 2]
            dst_q = dbufs[q % 2]
            if q + 1 < _NQ:
                start_idx(q + 1)

            def start_g(j, buf, sem, src_q=src_q):
                pltpu.async_copy(x_hbm.at[src_q.at[j]], buf, sem)

            def scat(j, buf, dst_q=dst_q):
                pltpu.sync_copy(buf, acc_s.at[dst_q.at[j]], add=True)

            start_g(0, rows0, semg0)

            def pair(i, carry, start_g=start_g, scat=scat):
                j0 = 2 * i
                start_g(j0 + 1, rows1, semg1)
                wait_g(rows0, semg0)
                scat(j0, rows0)

                @pl.when(i < _QPAIRS - 1)
                def _():
                    start_g(j0 + 2, rows0, semg0)

                wait_g(rows1, semg1)
                scat(j0 + 1, rows1)
                return carry

            lax.fori_loop(0, _QPAIRS, pair, None)
            if q + 1 < _NQ:
                wait_idx(q + 1)
        plsc.subcore_barrier()

        pltpu.sync_copy(
            acc_s.at[pl.ds(s * _RPS, _RPS)],
            out_hbm.at[c, pl.ds(s * _RPS, _RPS)],
        )

    return agg(x, src3, dst3)


def _tc_combine(partials, W, b2):
    bn = 1000
    grid = (_N // bn,)

    def body(p0_ref, p1_ref, w_ref, b_ref, o_ref):
        a = p0_ref[0] + p1_ref[0]
        h = lax.dot_general(a, w_ref[...], (((1,), (1,)), ((), ())),
                            preferred_element_type=jnp.float32)
        o_ref[...] = h + b_ref[...]

    return pl.pallas_call(
        body,
        grid=grid,
        in_specs=[
            pl.BlockSpec((1, bn, _D), lambda i: (0, i, 0)),
            pl.BlockSpec((1, bn, _D), lambda i: (1, i, 0)),
            pl.BlockSpec((_D, _D), lambda i: (0, 0)),
            pl.BlockSpec((1, _D), lambda i: (0, 0)),
        ],
        out_specs=pl.BlockSpec((bn, _D), lambda i: (i, 0)),
        out_shape=jax.ShapeDtypeStruct((_N, _D), jnp.float32),
    )(partials, partials, W, b2)


@jax.jit
def kernel(x, edge_index, W, b):
    src = edge_index[0]
    dst = edge_index[1]
    pad = _EPAD - _E
    src3 = jnp.concatenate(
        [src, jnp.zeros((pad,), jnp.int32)]).reshape(_NW * _NQ, _QCH, _CHUNK)
    dst3 = jnp.concatenate(
        [dst, jnp.full((pad,), _NPAD - 1, jnp.int32)]).reshape(_NW * _NQ, _QCH, _CHUNK)
    partials = _sc_aggregate(x, src3, dst3)
    out = _tc_combine(partials, W, b.reshape(1, _D))
    return (out,)


# speedup vs baseline: 1.0007x; 1.0007x over previous
import functools

import jax
import jax.numpy as jnp
from jax import lax
from jax.experimental import pallas as pl
from jax.experimental.pallas import tpu as pltpu
from jax.experimental.pallas import tpu_sc as plsc

_N = 10000
_E = 320000
_D = 128

_NC = 2
_NS = 16
_NW = _NC * _NS
_CHUNK = 128
_NCHUNK = 80
_NQ = 4
_QCH = _NCHUNK // _NQ
_QPAIRS = _QCH // 2
_EPW = _NCHUNK * _CHUNK
_EPAD = _NW * _EPW
_NPAD = 10240
_RPS = _NPAD // _NS


def _sc_aggregate(x, src3, dst3):
    mesh = plsc.VectorSubcoreMesh(core_axis_name="c", subcore_axis_name="s")

    @functools.partial(
        pl.kernel,
        mesh=mesh,
        out_type=jax.ShapeDtypeStruct((2, _NPAD, _D), jnp.float32),
        scratch_types=[
            pltpu.VMEM((_QCH, _CHUNK), jnp.int32),
            pltpu.VMEM((_QCH, _CHUNK), jnp.int32),
            pltpu.VMEM((_QCH, _CHUNK), jnp.int32),
            pltpu.VMEM((_QCH, _CHUNK), jnp.int32),
            pltpu.VMEM((_CHUNK, _D), jnp.float32),
            pltpu.VMEM((_CHUNK, _D), jnp.float32),
            pltpu.VMEM_SHARED((_NPAD, _D), jnp.float32),
            pltpu.SemaphoreType.DMA,
            pltpu.SemaphoreType.DMA,
            pltpu.SemaphoreType.DMA,
            pltpu.SemaphoreType.DMA,
        ],
    )
    def agg(x_hbm, src_hbm, dst_hbm, out_hbm, srcs0, srcs1, dsts0, dsts1,
            rows0, rows1, acc_s, semi0, semi1, semg0, semg1):
        c = lax.axis_index("c")
        s = lax.axis_index("s")
        wid = c * _NS + s
        sbufs = (srcs0, srcs1)
        dbufs = (dsts0, dsts1)
        isems = (semi0, semi1)

        def start_idx(q):
            b = q % 2
            pltpu.async_copy(src_hbm.at[wid * _NQ + q], sbufs[b], isems[b])
            pltpu.async_copy(dst_hbm.at[wid * _NQ + q], dbufs[b], isems[b])

        def wait_idx(q):
            b = q % 2
            pltpu.make_async_copy(src_hbm.at[wid * _NQ + q], sbufs[b], isems[b]).wait()
            pltpu.make_async_copy(dst_hbm.at[wid * _NQ + q], dbufs[b], isems[b]).wait()

        start_idx(0)

        def zstore(i, carry):
            rows0[i // (_D // 16), pl.ds((i % (_D // 16)) * 16, 16)] = jnp.zeros(
                (16,), jnp.float32)
            return carry

        lax.fori_loop(0, _CHUNK * (_D // 16), zstore, None)

        def zcopy(j, carry):
            pltpu.sync_copy(rows0, acc_s.at[pl.ds(s * _RPS + j * _CHUNK, _CHUNK)])
            return carry

        lax.fori_loop(0, _RPS // _CHUNK, zcopy, None)

        wait_idx(0)
        plsc.subcore_barrier()

        def wait_g(buf, sem):
            pltpu.make_async_copy(x_hbm.at[pl.ds(0, _CHUNK)], buf, sem).wait()

        for q in range(_NQ):
            src_q = sbufs[q % 2]
            dst_q = dbufs[q % 2]
            if q + 1 < _NQ:
                start_idx(q + 1)

            def start_g(j, buf, sem, src_q=src_q):
                pltpu.async_copy(x_hbm.at[src_q.at[j]], buf, sem)

            def scat(j, buf, dst_q=dst_q):
                pltpu.sync_copy(buf, acc_s.at[dst_q.at[j]], add=True)

            start_g(0, rows0, semg0)

            def pair(i, carry, start_g=start_g, scat=scat):
                j0 = 2 * i
                start_g(j0 + 1, rows1, semg1)
                wait_g(rows0, semg0)
                scat(j0, rows0)

                @pl.when(i < _QPAIRS - 1)
                def _():
                    start_g(j0 + 2, rows0, semg0)

                wait_g(rows1, semg1)
                scat(j0 + 1, rows1)
                return carry

            lax.fori_loop(0, _QPAIRS, pair, None)
            if q + 1 < _NQ:
                wait_idx(q + 1)
        plsc.subcore_barrier()

        pltpu.sync_copy(
            acc_s.at[pl.ds(s * _RPS, _RPS)],
            out_hbm.at[c, pl.ds(s * _RPS, _RPS)],
        )

    return agg(x, src3, dst3)


def _tc_combine(partials, W, b2):
    bn = 1000
    grid = (_N // bn,)

    def body(p0_ref, p1_ref, w_ref, b_ref, o_ref):
        a = p0_ref[0] + p1_ref[0]
        h = lax.dot_general(a, w_ref[...], (((1,), (1,)), ((), ())),
                            preferred_element_type=jnp.float32)
        o_ref[...] = h + b_ref[...]

    return pl.pallas_call(
        body,
        grid=grid,
        in_specs=[
            pl.BlockSpec((1, bn, _D), lambda i: (0, i, 0)),
            pl.BlockSpec((1, bn, _D), lambda i: (1, i, 0)),
            pl.BlockSpec((_D, _D), lambda i: (0, 0)),
            pl.BlockSpec((1, _D), lambda i: (0, 0)),
        ],
        out_specs=pl.BlockSpec((bn, _D), lambda i: (i, 0)),
        out_shape=jax.ShapeDtypeStruct((_N, _D), jnp.float32),
    )(partials, partials, W, b2)


@jax.jit
def kernel(x, edge_index, W, b):
    src = edge_index[0]
    dst = edge_index[1]
    pad = _EPAD - _E
    pad_dst = _N + (jnp.arange(pad, dtype=jnp.int32) % (_NPAD - _N))
    src3 = jnp.concatenate(
        [src, jnp.zeros((pad,), jnp.int32)]).reshape(_NW * _NQ, _QCH, _CHUNK)
    dst3 = jnp.concatenate(
        [dst, pad_dst]).reshape(_NW * _NQ, _QCH, _CHUNK)
    partials = _sc_aggregate(x, src3, dst3)
    out = _tc_combine(partials, W, b.reshape(1, _D))
    return (out,)


# speedup vs baseline: 2.5959x; 2.5941x over previous
import functools

import jax
import jax.numpy as jnp
from jax import lax
from jax.experimental import pallas as pl
from jax.experimental.pallas import tpu as pltpu
from jax.experimental.pallas import tpu_sc as plsc

_N = 10000
_E = 320000
_D = 128

_NC = 2
_NS = 16
_NW = _NC * _NS
_EPW = _E // _NW
_CHUNK = 80
_NCHUNK = _EPW // _CHUNK
_PAIRS = (_NCHUNK - 1) // 2
_NPAD = 10240
_RPS = _NPAD // _NS


def _sc_aggregate(x, src, dst):
    mesh = plsc.VectorSubcoreMesh(core_axis_name="c", subcore_axis_name="s")

    @functools.partial(
        pl.kernel,
        mesh=mesh,
        out_type=jax.ShapeDtypeStruct((2, _NPAD, _D), jnp.float32),
        scratch_types=[
            pltpu.VMEM((_CHUNK,), jnp.int32),
            pltpu.VMEM((_CHUNK,), jnp.int32),
            pltpu.VMEM((_CHUNK,), jnp.int32),
            pltpu.VMEM((_CHUNK,), jnp.int32),
            pltpu.VMEM((_CHUNK, _D), jnp.float32),
            pltpu.VMEM((_CHUNK, _D), jnp.float32),
            pltpu.VMEM_SHARED((_NPAD, _D), jnp.float32),
            pltpu.SemaphoreType.DMA,
            pltpu.SemaphoreType.DMA,
        ],
    )
    def agg(x_hbm, src_hbm, dst_hbm, out_hbm, srcA, srcB, dstA, dstB,
            rows0, rows1, acc_s, semg0, semg1):
        c = lax.axis_index("c")
        s = lax.axis_index("s")
        wid = c * _NS + s
        ebase = wid * _EPW

        def sync_idx(j, sbuf, dbuf):
            pltpu.sync_copy(src_hbm.at[pl.ds(ebase + j * _CHUNK, _CHUNK)], sbuf)
            pltpu.sync_copy(dst_hbm.at[pl.ds(ebase + j * _CHUNK, _CHUNK)], dbuf)

        def start_g(sbuf, buf, sem):
            pltpu.async_copy(x_hbm.at[sbuf], buf, sem)

        def wait_g(buf, sem):
            pltpu.make_async_copy(x_hbm.at[pl.ds(0, _CHUNK)], buf, sem).wait()

        def scat(dbuf, buf):
            pltpu.sync_copy(buf, acc_s.at[dbuf], add=True)

        def zstore(i, carry):
            rows0[i // (_D // 16), pl.ds((i % (_D // 16)) * 16, 16)] = jnp.zeros(
                (16,), jnp.float32)
            return carry

        lax.fori_loop(0, _CHUNK * (_D // 16), zstore, None)

        def zcopy(j, carry):
            pltpu.sync_copy(rows0, acc_s.at[pl.ds(s * _RPS + j * _CHUNK, _CHUNK)])
            return carry

        lax.fori_loop(0, _RPS // _CHUNK, zcopy, None)
        plsc.subcore_barrier()

        sync_idx(0, srcA, dstA)
        start_g(srcA, rows0, semg0)

        def pair(i, carry):
            j0 = 2 * i
            sync_idx(j0 + 1, srcB, dstB)
            start_g(srcB, rows1, semg1)
            wait_g(rows0, semg0)
            scat(dstA, rows0)
            sync_idx(j0 + 2, srcA, dstA)
            start_g(srcA, rows0, semg0)
            wait_g(rows1, semg1)
            scat(dstB, rows1)
            return carry

        lax.fori_loop(0, _PAIRS, pair, None)

        wait_g(rows0, semg0)
        scat(dstA, rows0)
        plsc.subcore_barrier()

        pltpu.sync_copy(
            acc_s.at[pl.ds(s * _RPS, _RPS)],
            out_hbm.at[c, pl.ds(s * _RPS, _RPS)],
        )

    return agg(x, src, dst)


def _tc_combine(partials, W, b2):
    bn = 1000
    grid = (_N // bn,)

    def body(p0_ref, p1_ref, w_ref, b_ref, o_ref):
        a = p0_ref[0] + p1_ref[0]
        h = lax.dot_general(a, w_ref[...], (((1,), (1,)), ((), ())),
                            preferred_element_type=jnp.float32)
        o_ref[...] = h + b_ref[...]

    return pl.pallas_call(
        body,
        grid=grid,
        in_specs=[
            pl.BlockSpec((1, bn, _D), lambda i: (0, i, 0)),
            pl.BlockSpec((1, bn, _D), lambda i: (1, i, 0)),
            pl.BlockSpec((_D, _D), lambda i: (0, 0)),
            pl.BlockSpec((1, _D), lambda i: (0, 0)),
        ],
        out_specs=pl.BlockSpec((bn, _D), lambda i: (i, 0)),
        out_shape=jax.ShapeDtypeStruct((_N, _D), jnp.float32),
    )(partials, partials, W, b2)


@jax.jit
def kernel(x, edge_index, W, b):
    src = edge_index[0]
    dst = edge_index[1]
    partials = _sc_aggregate(x, src, dst)
    out = _tc_combine(partials, W, b.reshape(1, _D))
    return (out,)
